# baseline (speedup 1.0000x reference)
import functools
import jax
import jax.numpy as jnp
from jax import lax
from jax.experimental import pallas as pl
from jax.experimental.pallas import tpu as pltpu
from jax.experimental.pallas import tpu_sc as plsc

B = 2048
H = 1024
HH = 512
E = 16
OUT = 16
EO = E * OUT

BB = 512
NC = 2
NS = 16
NW = NC * NS
L = 16
TPW = B // NW
NG = TPW // L


def _tc_dense_kernel(q_ref, x_ref, W1_ref, b1_ref, W2_ref, b2_ref,
                     WeT_ref, beR_ref, logits_ref, y_ref):
    h = jnp.maximum(jnp.dot(q_ref[...], W1_ref[...],
                            preferred_element_type=jnp.float32) + b1_ref[...], 0.0)
    logits_ref[...] = jnp.dot(h, W2_ref[...],
                              preferred_element_type=jnp.float32) + b2_ref[...]
    y_ref[...] = jnp.dot(x_ref[...], WeT_ref[...],
                         preferred_element_type=jnp.float32) + beR_ref[...]


def _tc_dense(q, x, W1, b1, W2, b2, WeT, beR):
    grid = (B // BB,)
    return pl.pallas_call(
        _tc_dense_kernel,
        grid=grid,
        in_specs=[
            pl.BlockSpec((BB, H), lambda i: (i, 0)),
            pl.BlockSpec((BB, H), lambda i: (i, 0)),
            pl.BlockSpec((H, HH), lambda i: (0, 0)),
            pl.BlockSpec((HH,), lambda i: (0,)),
            pl.BlockSpec((HH, E), lambda i: (0, 0)),
            pl.BlockSpec((E,), lambda i: (0,)),
            pl.BlockSpec((H, EO), lambda i: (0, 0)),
            pl.BlockSpec((1, EO), lambda i: (0, 0)),
        ],
        out_specs=[
            pl.BlockSpec((BB, E), lambda i: (i, 0)),
            pl.BlockSpec((BB, EO), lambda i: (i, 0)),
        ],
        out_shape=[
            jax.ShapeDtypeStruct((B, E), jnp.float32),
            jax.ShapeDtypeStruct((B, EO), jnp.float32),
        ],
    )(q, x, W1, b1, W2, b2, WeT, beR)


def _sc_routing_body(logits_hbm, y_hbm, out_hbm, lg_v, y_v, o_v):
    wid = lax.axis_index("s") * NC + lax.axis_index("c")
    base = wid * TPW
    pltpu.sync_copy(logits_hbm.at[pl.ds(base * E, TPW * E)], lg_v)
    pltpu.sync_copy(y_hbm.at[pl.ds(base * EO, TPW * EO)], y_v)

    lanes = lax.iota(jnp.int32, L)
    for g in range(NG):
        rowidx = g * L + lanes
        m0 = jnp.full((L,), -jnp.inf, jnp.float32)
        m1 = jnp.full((L,), -jnp.inf, jnp.float32)
        i0 = jnp.zeros((L,), jnp.int32)
        i1 = jnp.zeros((L,), jnp.int32)
        lgbase = rowidx * E
        for e in range(E):
            v = plsc.load_gather(lg_v, [lgbase + e])
            is0 = v > m0
            is1 = jnp.logical_and(jnp.logical_not(is0), v > m1)
            m1 = jnp.where(is0, m0, jnp.where(is1, v, m1))
            i1 = jnp.where(is0, i0, jnp.where(is1, e, i1))
            m0 = jnp.where(is0, v, m0)
            i0 = jnp.where(is0, e, i0)
        ex = jnp.exp(m1 - m0)
        g0 = 1.0 / (1.0 + ex)
        g1 = ex * g0
        c0 = rowidx * EO + i0 * OUT
        c1 = rowidx * EO + i1 * OUT
        obase = rowidx * OUT
        for o in range(OUT):
            y0 = plsc.load_gather(y_v, [c0 + o])
            y1 = plsc.load_gather(y_v, [c1 + o])
            plsc.store_scatter(o_v, [obase + o], g0 * y0 + g1 * y1)

    pltpu.sync_copy(o_v, out_hbm.at[pl.ds(base * OUT, TPW * OUT)])


_sc_routing = functools.partial(
    pl.kernel,
    mesh=plsc.VectorSubcoreMesh(core_axis_name="c", subcore_axis_name="s",
                                num_cores=NC, num_subcores=NS),
    compiler_params=pltpu.CompilerParams(needs_layout_passes=False),
    out_type=jax.ShapeDtypeStruct((B * OUT,), jnp.float32),
    scratch_types=[
        pltpu.VMEM((TPW * E,), jnp.float32),
        pltpu.VMEM((TPW * EO,), jnp.float32),
        pltpu.VMEM((TPW * OUT,), jnp.float32),
    ],
)(_sc_routing_body)


def kernel(query_repr, x, W1, b1, W2, b2, We, be):
    WeT = We.reshape(EO, H).T
    beR = be.reshape(1, EO)
    logits, y = _tc_dense(query_repr, x, W1, b1, W2, b2, WeT, beR)
    out_flat = _sc_routing(logits.reshape(B * E), y.reshape(B * EO))
    return out_flat.reshape(B, OUT)


# speedup vs baseline: 1.0997x; 1.0997x over previous
import functools
import jax
import jax.numpy as jnp
from jax import lax
from jax.experimental import pallas as pl
from jax.experimental.pallas import tpu as pltpu
from jax.experimental.pallas import tpu_sc as plsc

B = 2048
H = 1024
HH = 512
E = 16
OUT = 16
EO = E * OUT

BB = 512
NC = 2
NS = 16
NW = NC * NS
L = 16
TPW = B // NW
NG = TPW // L


def _tc_dense_kernel(q_ref, x_ref, W1_ref, b1_ref, W2_ref, b2_ref,
                     WeT_ref, beR_ref, logits_ref, y_ref):
    h = jnp.maximum(jnp.dot(q_ref[...], W1_ref[...],
                            preferred_element_type=jnp.float32) + b1_ref[...], 0.0)
    logits_ref[...] = jnp.dot(h, W2_ref[...],
                              preferred_element_type=jnp.float32) + b2_ref[...]
    y_ref[...] = jnp.dot(x_ref[...], WeT_ref[...],
                         preferred_element_type=jnp.float32) + beR_ref[...]


def _tc_dense(q, x, W1, b1, W2, b2, WeT, beR):
    grid = (B // BB,)
    return pl.pallas_call(
        _tc_dense_kernel,
        grid=grid,
        in_specs=[
            pl.BlockSpec((BB, H), lambda i: (i, 0)),
            pl.BlockSpec((BB, H), lambda i: (i, 0)),
            pl.BlockSpec((H, HH), lambda i: (0, 0)),
            pl.BlockSpec((HH,), lambda i: (0,)),
            pl.BlockSpec((HH, E), lambda i: (0, 0)),
            pl.BlockSpec((E,), lambda i: (0,)),
            pl.BlockSpec((H, EO), lambda i: (0, 0)),
            pl.BlockSpec((1, EO), lambda i: (0, 0)),
        ],
        out_specs=[
            pl.BlockSpec((BB, E), lambda i: (i, 0)),
            pl.BlockSpec((BB, EO), lambda i: (i, 0)),
        ],
        out_shape=[
            jax.ShapeDtypeStruct((B, E), jnp.float32),
            jax.ShapeDtypeStruct((B, EO), jnp.float32),
        ],
    )(q, x, W1, b1, W2, b2, WeT, beR)


def _sc_routing_body(logits_hbm, y_hbm, out_hbm, lg_v, y_v, o_v):
    wid = lax.axis_index("s") * NC + lax.axis_index("c")
    base = wid * TPW
    pltpu.sync_copy(logits_hbm.at[pl.ds(base, TPW)], lg_v)
    pltpu.sync_copy(y_hbm.at[pl.ds(base, TPW)], y_v)

    lanes = lax.iota(jnp.int32, L)
    for g in range(NG):
        rowidx = g * L + lanes
        m0 = jnp.full((L,), -jnp.inf, jnp.float32)
        m1 = jnp.full((L,), -jnp.inf, jnp.float32)
        i0 = jnp.zeros((L,), jnp.int32)
        i1 = jnp.zeros((L,), jnp.int32)
        for e in range(E):
            v = plsc.load_gather(lg_v, [rowidx, jnp.full((L,), e, jnp.int32)])
            is0 = v > m0
            is1 = jnp.logical_and(jnp.logical_not(is0), v > m1)
            m1 = jnp.where(is0, m0, jnp.where(is1, v, m1))
            i1 = jnp.where(is0, i0, jnp.where(is1, e, i1))
            m0 = jnp.where(is0, v, m0)
            i0 = jnp.where(is0, e, i0)
        ex = jnp.exp(m1 - m0)
        g0 = 1.0 / (1.0 + ex)
        g1 = ex * g0
        c0 = i0 * OUT
        c1 = i1 * OUT
        for o in range(OUT):
            y0 = plsc.load_gather(y_v, [rowidx, c0 + o])
            y1 = plsc.load_gather(y_v, [rowidx, c1 + o])
            plsc.store_scatter(o_v, [rowidx, jnp.full((L,), o, jnp.int32)],
                               g0 * y0 + g1 * y1)

    pltpu.sync_copy(o_v, out_hbm.at[pl.ds(base, TPW)])


_sc_routing = functools.partial(
    pl.kernel,
    mesh=plsc.VectorSubcoreMesh(core_axis_name="c", subcore_axis_name="s",
                                num_cores=NC, num_subcores=NS),
    compiler_params=pltpu.CompilerParams(needs_layout_passes=False),
    out_type=jax.ShapeDtypeStruct((B, OUT), jnp.float32),
    scratch_types=[
        pltpu.VMEM((TPW, E), jnp.float32),
        pltpu.VMEM((TPW, EO), jnp.float32),
        pltpu.VMEM((TPW, OUT), jnp.float32),
    ],
)(_sc_routing_body)


def kernel(query_repr, x, W1, b1, W2, b2, We, be):
    WeT = We.reshape(EO, H).T
    beR = be.reshape(1, EO)
    logits, y = _tc_dense(query_repr, x, W1, b1, W2, b2, WeT, beR)
    return _sc_routing(logits, y)
